# baseline (speedup 1.0000x reference)
import jax
import jax.numpy as jnp
from jax.experimental import pallas as pl
from jax.experimental.pallas import tpu as pltpu

NQ = 64
NG = 192
C = 128
HW = 32


def _qga_kernel(x_ref, thw_ref, thb_ref, phw_ref, phb_ref, out_ref,
                q_scr, b_scr):
    i = pl.program_id(0)

    @pl.when(i == 0)
    def _():
        xqt = x_ref[pl.ds(0, NQ)].reshape(NQ * HW, C)
        p = jnp.dot(
            xqt, phw_ref[...], preferred_element_type=jnp.float32
        ) + phb_ref[...]
        q_scr[...] = jax.lax.dot_general(
            p, thw_ref[...], (((1,), (1,)), ((), ())),
            preferred_element_type=jnp.float32,
        ).astype(jnp.bfloat16)
        b_scr[...] = jax.lax.dot_general(
            p, thb_ref[...], (((1,), (1,)), ((), ())),
            preferred_element_type=jnp.float32,
        )

    xgi = x_ref[pl.ds(NQ, NG), pl.ds(i, 1), :].reshape(NG, C)
    m = (jax.lax.dot_general(
        q_scr[...], xgi.astype(jnp.bfloat16), (((1,), (1,)), ((), ())),
        preferred_element_type=jnp.float32,
    ) + b_scr[...]) * (1.0 / HW)
    f = jnp.max(m.reshape(NQ, HW, NG), axis=1)
    out_ref[...] = jax.nn.sigmoid(f)[None]


def kernel(x, num_query, theta_w, theta_b, phi_w, phi_b):
    del num_query
    xt = jnp.transpose(x, (0, 2, 3, 1)).reshape(x.shape[0], HW, C)
    out = pl.pallas_call(
        _qga_kernel,
        grid=(HW,),
        in_specs=[
            pl.BlockSpec((xt.shape[0], HW, C), lambda i: (0, 0, 0)),
            pl.BlockSpec((C, C), lambda i: (0, 0)),
            pl.BlockSpec((1, C), lambda i: (0, 0)),
            pl.BlockSpec((C, C), lambda i: (0, 0)),
            pl.BlockSpec((1, C), lambda i: (0, 0)),
        ],
        out_specs=pl.BlockSpec((1, NQ, NG), lambda i: (i, 0, 0)),
        out_shape=jax.ShapeDtypeStruct((HW, NQ, NG), jnp.float32),
        scratch_shapes=[pltpu.VMEM((NQ * HW, C), jnp.bfloat16),
                        pltpu.VMEM((NQ * HW, 1), jnp.float32)],
    )(xt, theta_w.T, theta_b.reshape(1, C), phi_w.T, phi_b.reshape(1, C))
    return jnp.transpose(out, (1, 2, 0)).reshape(
        NQ * NG, 1, x.shape[2], x.shape[3])


# speedup vs baseline: 1.5235x; 1.5235x over previous
import jax
import jax.numpy as jnp
from jax.experimental import pallas as pl
from jax.experimental.pallas import tpu as pltpu

NQ = 64
NG = 192
C = 128
HW = 32
PX = 4


def _qga_kernel(x_ref, thw_ref, thb_ref, phw_ref, phb_ref, out_ref,
                q_scr, b_scr):
    i = pl.program_id(0)

    @pl.when(i == 0)
    def _():
        xqt = x_ref[pl.ds(0, NQ)].reshape(NQ * HW, C)
        p = jnp.dot(
            xqt, phw_ref[...], preferred_element_type=jnp.float32
        ) + phb_ref[...]
        q_scr[...] = jax.lax.dot_general(
            p, thw_ref[...], (((1,), (1,)), ((), ())),
            preferred_element_type=jnp.float32,
        ).astype(jnp.bfloat16)
        b_scr[...] = jax.lax.dot_general(
            p, thb_ref[...], (((1,), (1,)), ((), ())),
            preferred_element_type=jnp.float32,
        )

    xgi = jnp.concatenate(
        [x_ref[pl.ds(NQ, NG), pl.ds(i * PX + k, 1), :].reshape(NG, C)
         for k in range(PX)], axis=0).astype(jnp.bfloat16)
    m = jax.lax.dot_general(
        q_scr[...], xgi, (((1,), (1,)), ((), ())),
        preferred_element_type=jnp.float32,
    ) + b_scr[...]
    f = jnp.max(m.reshape(NQ, HW, PX * NG), axis=1)
    f = jax.nn.sigmoid(f * (1.0 / HW))
    for k in range(PX):
        out_ref[k] = f[:, k * NG:(k + 1) * NG]


def kernel(x, num_query, theta_w, theta_b, phi_w, phi_b):
    del num_query
    xt = jnp.transpose(x, (0, 2, 3, 1)).reshape(x.shape[0], HW, C)
    out = pl.pallas_call(
        _qga_kernel,
        grid=(HW // PX,),
        in_specs=[
            pl.BlockSpec((xt.shape[0], HW, C), lambda i: (0, 0, 0)),
            pl.BlockSpec((C, C), lambda i: (0, 0)),
            pl.BlockSpec((1, C), lambda i: (0, 0)),
            pl.BlockSpec((C, C), lambda i: (0, 0)),
            pl.BlockSpec((1, C), lambda i: (0, 0)),
        ],
        out_specs=pl.BlockSpec((PX, NQ, NG), lambda i: (i, 0, 0)),
        out_shape=jax.ShapeDtypeStruct((HW, NQ, NG), jnp.float32),
        scratch_shapes=[pltpu.VMEM((NQ * HW, C), jnp.bfloat16),
                        pltpu.VMEM((NQ * HW, 1), jnp.float32)],
    )(xt, theta_w.T, theta_b.reshape(1, C), phi_w.T, phi_b.reshape(1, C))
    return jnp.transpose(out, (1, 2, 0)).reshape(
        NQ * NG, 1, x.shape[2], x.shape[3])


# speedup vs baseline: 1.5473x; 1.0156x over previous
import jax
import jax.numpy as jnp
from jax.experimental import pallas as pl
from jax.experimental.pallas import tpu as pltpu

NQ = 64
NG = 192
C = 128
HW = 32
PX = 8


def _qga_kernel(x_ref, thw_ref, thb_ref, phw_ref, phb_ref, out_ref,
                q_scr, b_scr):
    i = pl.program_id(0)

    @pl.when(i == 0)
    def _():
        xqt = x_ref[pl.ds(0, NQ)].reshape(NQ * HW, C)
        p = jnp.dot(
            xqt, phw_ref[...], preferred_element_type=jnp.float32
        ) + phb_ref[...]
        q_scr[...] = jax.lax.dot_general(
            p, thw_ref[...], (((1,), (1,)), ((), ())),
            preferred_element_type=jnp.float32,
        ).astype(jnp.bfloat16)
        b_scr[...] = jax.lax.dot_general(
            p, thb_ref[...], (((1,), (1,)), ((), ())),
            preferred_element_type=jnp.float32,
        )

    xgi = jnp.concatenate(
        [x_ref[pl.ds(NQ, NG), pl.ds(i * PX + k, 1), :].reshape(NG, C)
         for k in range(PX)], axis=0).astype(jnp.bfloat16)
    m = jax.lax.dot_general(
        q_scr[...], xgi, (((1,), (1,)), ((), ())),
        preferred_element_type=jnp.float32,
    ) + b_scr[...]
    f = jnp.max(m.reshape(NQ, HW, PX * NG), axis=1)
    f = jax.nn.sigmoid(f * (1.0 / HW))
    for k in range(PX):
        out_ref[k] = f[:, k * NG:(k + 1) * NG]


def kernel(x, num_query, theta_w, theta_b, phi_w, phi_b):
    del num_query
    xt = jnp.transpose(x, (0, 2, 3, 1)).reshape(x.shape[0], HW, C)
    out = pl.pallas_call(
        _qga_kernel,
        grid=(HW // PX,),
        in_specs=[
            pl.BlockSpec((xt.shape[0], HW, C), lambda i: (0, 0, 0)),
            pl.BlockSpec((C, C), lambda i: (0, 0)),
            pl.BlockSpec((1, C), lambda i: (0, 0)),
            pl.BlockSpec((C, C), lambda i: (0, 0)),
            pl.BlockSpec((1, C), lambda i: (0, 0)),
        ],
        out_specs=pl.BlockSpec((PX, NQ, NG), lambda i: (i, 0, 0)),
        out_shape=jax.ShapeDtypeStruct((HW, NQ, NG), jnp.float32),
        scratch_shapes=[pltpu.VMEM((NQ * HW, C), jnp.bfloat16),
                        pltpu.VMEM((NQ * HW, 1), jnp.float32)],
    )(xt, theta_w.T, theta_b.reshape(1, C), phi_w.T, phi_b.reshape(1, C))
    return jnp.transpose(out, (1, 2, 0)).reshape(
        NQ * NG, 1, x.shape[2], x.shape[3])


# speedup vs baseline: 1.8105x; 1.1701x over previous
import jax
import jax.numpy as jnp
from jax.experimental import pallas as pl
from jax.experimental.pallas import tpu as pltpu

NQ = 64
NG = 192
C = 128
HW = 32
PX = 8


def _qga_kernel(x_ref, thw_ref, thb_ref, phw_ref, phb_ref, out_ref,
                q_scr, b_scr):
    i = pl.program_id(0)

    @pl.when(i == 0)
    def _():
        xqt = x_ref[pl.ds(0, NQ)].reshape(NQ * HW, C)
        p = jax.lax.dot_general(
            xqt, phw_ref[...], (((1,), (1,)), ((), ())),
            preferred_element_type=jnp.float32,
        ) + phb_ref[...]
        q_scr[...] = jax.lax.dot_general(
            p, thw_ref[...], (((1,), (0,)), ((), ())),
            preferred_element_type=jnp.float32,
        ).astype(jnp.bfloat16)
        b_scr[...] = jax.lax.dot_general(
            p, thb_ref[...], (((1,), (1,)), ((), ())),
            preferred_element_type=jnp.float32,
        )

    xgi = jnp.concatenate(
        [x_ref[pl.ds(NQ, NG), pl.ds(i * PX + k, 1), :].reshape(NG, C)
         for k in range(PX)], axis=0).astype(jnp.bfloat16)
    m = jax.lax.dot_general(
        q_scr[...], xgi, (((1,), (1,)), ((), ())),
        preferred_element_type=jnp.float32,
    ) + b_scr[...]
    f = jnp.max(m.reshape(NQ, HW, PX * NG), axis=1)
    f = jax.nn.sigmoid(f * (1.0 / HW))
    for k in range(PX):
        out_ref[k] = f[:, k * NG:(k + 1) * NG]


def kernel(x, num_query, theta_w, theta_b, phi_w, phi_b):
    del num_query
    xt = jnp.transpose(x, (0, 2, 3, 1)).reshape(x.shape[0], HW, C)
    out = pl.pallas_call(
        _qga_kernel,
        grid=(HW // PX,),
        in_specs=[
            pl.BlockSpec((xt.shape[0], HW, C), lambda i: (0, 0, 0)),
            pl.BlockSpec((C, C), lambda i: (0, 0)),
            pl.BlockSpec((1, C), lambda i: (0, 0)),
            pl.BlockSpec((C, C), lambda i: (0, 0)),
            pl.BlockSpec((1, C), lambda i: (0, 0)),
        ],
        out_specs=pl.BlockSpec((PX, NQ, NG), lambda i: (i, 0, 0)),
        out_shape=jax.ShapeDtypeStruct((HW, NQ, NG), jnp.float32),
        scratch_shapes=[pltpu.VMEM((NQ * HW, C), jnp.bfloat16),
                        pltpu.VMEM((NQ * HW, 1), jnp.float32)],
    )(xt, theta_w, theta_b.reshape(1, C), phi_w, phi_b.reshape(1, C))
    return jnp.transpose(out, (1, 2, 0)).reshape(
        NQ * NG, 1, x.shape[2], x.shape[3])


# speedup vs baseline: 1.8573x; 1.0259x over previous
import jax
import jax.numpy as jnp
from jax.experimental import pallas as pl
from jax.experimental.pallas import tpu as pltpu

NQ = 64
NG = 192
C = 128
HW = 32
PX = 8
CH = 4


def _qga_kernel(x_ref, thw_ref, thb_ref, phw_ref, phb_ref, out_ref, q_scr):
    i = pl.program_id(0)

    @pl.when(i == 0)
    def _():
        xqt = x_ref[pl.ds(0, NQ)].reshape(NQ * HW, C)
        p = jax.lax.dot_general(
            xqt, phw_ref[...], (((1,), (1,)), ((), ())),
            preferred_element_type=jnp.float32,
        ) + phb_ref[...]
        q = jax.lax.dot_general(
            p, thw_ref[...], (((1,), (0,)), ((), ())),
            preferred_element_type=jnp.float32,
        )
        b = jax.lax.dot_general(
            p, thb_ref[...], (((1,), (1,)), ((), ())),
            preferred_element_type=jnp.float32,
        )
        q_scr[...] = jnp.concatenate(
            [q, jnp.broadcast_to(b, (NQ * HW, C))], axis=1
        ).astype(jnp.bfloat16)

    ones_col = (jax.lax.broadcasted_iota(jnp.int32, (CH * NG, C), 1)
                == 0).astype(jnp.bfloat16)
    for c in range(PX // CH):
        xg_part = jnp.concatenate(
            [x_ref[pl.ds(NQ, NG), pl.ds(i * PX + c * CH + k, 1), :]
             .reshape(NG, C).astype(jnp.bfloat16) for k in range(CH)],
            axis=0)
        xgi = jnp.concatenate([xg_part, ones_col], axis=1)
        m = jax.lax.dot_general(
            q_scr[...], xgi, (((1,), (1,)), ((), ())),
            preferred_element_type=jnp.float32,
        )
        f = jnp.max(m.reshape(NQ, HW, CH * NG), axis=1)
        f = jax.nn.sigmoid(f * (1.0 / HW))
        for k in range(CH):
            out_ref[c * CH + k] = f[:, k * NG:(k + 1) * NG]


def kernel(x, num_query, theta_w, theta_b, phi_w, phi_b):
    del num_query
    xt = jnp.transpose(x, (0, 2, 3, 1)).reshape(x.shape[0], HW, C)
    out = pl.pallas_call(
        _qga_kernel,
        grid=(HW // PX,),
        in_specs=[
            pl.BlockSpec((xt.shape[0], HW, C), lambda i: (0, 0, 0)),
            pl.BlockSpec((C, C), lambda i: (0, 0)),
            pl.BlockSpec((1, C), lambda i: (0, 0)),
            pl.BlockSpec((C, C), lambda i: (0, 0)),
            pl.BlockSpec((1, C), lambda i: (0, 0)),
        ],
        out_specs=pl.BlockSpec((PX, NQ, NG), lambda i: (i, 0, 0)),
        out_shape=jax.ShapeDtypeStruct((HW, NQ, NG), jnp.float32),
        scratch_shapes=[pltpu.VMEM((NQ * HW, 2 * C), jnp.bfloat16)],
    )(xt, theta_w, theta_b.reshape(1, C), phi_w, phi_b.reshape(1, C))
    return jnp.transpose(out, (1, 2, 0)).reshape(
        NQ * NG, 1, x.shape[2], x.shape[3])
